# baseline (speedup 1.0000x reference)
import functools

import jax
import jax.numpy as jnp
from jax import lax
from jax.experimental import pallas as pl
from jax.experimental.pallas import tpu as pltpu
from jax.experimental.pallas import tpu_sc as plsc

N = 10000
NP = 10240
E = 320000
D = 128
H = 64
C = 40

NSUB = 16
NCORE = 2
CH = 128
NCHUNK = 160
EPT = NCHUNK * CH
EPG = NSUB * EPT
RPT = NP // NSUB

_f32 = jnp.float32
_i32 = jnp.int32

_MESH = plsc.VectorSubcoreMesh(core_axis_name="c", subcore_axis_name="s",
                               num_cores=NCORE, num_subcores=NSUB)


def _sds(shape, dtype=_f32):
    return jax.ShapeDtypeStruct(shape, dtype)


@functools.partial(
    pl.kernel,
    out_type=[pltpu.HBM((NP, 16), _f32)] * 2,
    mesh=_MESH,
    compiler_params=pltpu.CompilerParams(use_tc_tiling_on_sc=False,
                                         needs_layout_passes=False),
    scratch_types=[
        pltpu.VMEM((NCHUNK, CH), _i32),
        pltpu.VMEM((NCHUNK, CH), _i32),
        pltpu.VMEM((CH, 16), _f32),
        pltpu.VMEM((CH, 16), _f32),
        pltpu.VMEM((RPT, 16), _f32),
        pltpu.VMEM_SHARED((NP, 16), _f32),
    ],
)
def _deg_kernel(src0, dst0, src1, dst1, dtab0, dtab1,
                sbuf, dbuf, ones_s, ones_d, obuf, acc):
    cid = lax.axis_index("c")
    sid = lax.axis_index("s")

    @pl.when(cid == 0)
    def _():
        pltpu.sync_copy(src0.at[sid], sbuf)
        pltpu.sync_copy(dst0.at[sid], dbuf)

    @pl.when(cid == 1)
    def _():
        pltpu.sync_copy(src1.at[sid], sbuf)
        pltpu.sync_copy(dst1.at[sid], dbuf)

    lanes = lax.iota(_i32, 16)
    pat_s = jnp.where(lanes < 8, 1.0, 0.0).astype(_f32)
    pat_d = jnp.where(lanes < 8, 0.0, 1.0).astype(_f32)
    zero = jnp.zeros((16,), _f32)

    def init_ones(i, carry):
        ones_s[i, :] = pat_s
        ones_d[i, :] = pat_d
        return carry
    lax.fori_loop(0, CH, init_ones, 0)

    def init_zero(i, carry):
        obuf[i, :] = zero
        return carry
    lax.fori_loop(0, RPT, init_zero, 0)

    base = sid * RPT
    pltpu.sync_copy(obuf, acc.at[pl.ds(base, RPT)])
    plsc.subcore_barrier()

    def chunk(ci, carry):
        pltpu.sync_copy(ones_s, acc.at[sbuf.at[ci]], add=True)
        pltpu.sync_copy(ones_d, acc.at[dbuf.at[ci]], add=True)
        return carry
    lax.fori_loop(0, NCHUNK, chunk, 0)
    plsc.subcore_barrier()

    pltpu.sync_copy(acc.at[pl.ds(base, RPT)], obuf)

    @pl.when(cid == 0)
    def _():
        pltpu.sync_copy(obuf, dtab0.at[pl.ds(base, RPT)])

    @pl.when(cid == 1)
    def _():
        pltpu.sync_copy(obuf, dtab1.at[pl.ds(base, RPT)])


@functools.partial(
    pl.kernel,
    out_type=[pltpu.HBM((NSUB, NCHUNK, CH), _f32)] * 2,
    mesh=_MESH,
    compiler_params=pltpu.CompilerParams(use_tc_tiling_on_sc=False,
                                         needs_layout_passes=False),
    scratch_types=[
        pltpu.VMEM((NP,), _f32),
        pltpu.VMEM((NP,), _f32),
        pltpu.VMEM((NCHUNK, CH), _i32),
        pltpu.VMEM((NCHUNK, CH), _i32),
        pltpu.VMEM((NCHUNK, CH), _f32),
    ],
)
def _fold_kernel(src0, dst0, w0, src1, dst1, w1, ns0, nd0, ns1, nd1,
                 wp0, wp1, ns_t, nd_t, sbuf, dbuf, wbuf):
    cid = lax.axis_index("c")
    sid = lax.axis_index("s")

    @pl.when(cid == 0)
    def _():
        pltpu.sync_copy(ns0, ns_t)
        pltpu.sync_copy(nd0, nd_t)
        pltpu.sync_copy(src0.at[sid], sbuf)
        pltpu.sync_copy(dst0.at[sid], dbuf)
        pltpu.sync_copy(w0.at[sid], wbuf)

    @pl.when(cid == 1)
    def _():
        pltpu.sync_copy(ns1, ns_t)
        pltpu.sync_copy(nd1, nd_t)
        pltpu.sync_copy(src1.at[sid], sbuf)
        pltpu.sync_copy(dst1.at[sid], dbuf)
        pltpu.sync_copy(w1.at[sid], wbuf)

    def chunk(ci, carry):
        def group(gi, c2):
            sv = sbuf[ci, pl.ds(gi * 16, 16)]
            dv = dbuf[ci, pl.ds(gi * 16, 16)]
            wv = wbuf[ci, pl.ds(gi * 16, 16)]
            nsv = plsc.load_gather(ns_t, [sv])
            ndv = plsc.load_gather(nd_t, [dv])
            wbuf[ci, pl.ds(gi * 16, 16)] = wv * nsv * ndv
            return c2
        lax.fori_loop(0, CH // 16, group, 0)
        return carry
    lax.fori_loop(0, NCHUNK, chunk, 0)

    @pl.when(cid == 0)
    def _():
        pltpu.sync_copy(wbuf, wp0.at[sid])

    @pl.when(cid == 1)
    def _():
        pltpu.sync_copy(wbuf, wp1.at[sid])


@functools.partial(
    pl.kernel,
    out_type=[pltpu.HBM((NP, H), _f32)] * 2,
    mesh=_MESH,
    compiler_params=pltpu.CompilerParams(use_tc_tiling_on_sc=False,
                                         needs_layout_passes=False),
    scratch_types=[
        pltpu.VMEM((NCHUNK, CH), _i32),
        pltpu.VMEM((NCHUNK, CH), _i32),
        pltpu.VMEM((NCHUNK, CH), _f32),
        pltpu.VMEM((CH, H), _f32),
        pltpu.VMEM((CH, H), _f32),
        pltpu.VMEM((CH, H), _f32),
        pltpu.VMEM_SHARED((NP, H), _f32),
        pltpu.SemaphoreType.DMA,
        pltpu.SemaphoreType.DMA,
        pltpu.SemaphoreType.DMA,
        pltpu.SemaphoreType.DMA,
        pltpu.SemaphoreType.DMA,
        pltpu.SemaphoreType.DMA,
    ],
)
def _conv_kernel(y, src0, dst0, w0, src1, dst1, w1,
                 out0, out1,
                 sbuf, dbuf, wbuf, rows0, rows1, rows2, acc,
                 gsem0, gsem1, gsem2, ssem0, ssem1, ssem2):
    cid = lax.axis_index("c")
    sid = lax.axis_index("s")

    @pl.when(cid == 0)
    def _():
        pltpu.sync_copy(src0.at[sid], sbuf)
        pltpu.sync_copy(dst0.at[sid], dbuf)
        pltpu.sync_copy(w0.at[sid], wbuf)

    @pl.when(cid == 1)
    def _():
        pltpu.sync_copy(src1.at[sid], sbuf)
        pltpu.sync_copy(dst1.at[sid], dbuf)
        pltpu.sync_copy(w1.at[sid], wbuf)

    zero = jnp.zeros((16,), _f32)

    def init_zero(i, carry):
        for j in range(H // 16):
            rows0[i, pl.ds(j * 16, 16)] = zero
        return carry
    lax.fori_loop(0, CH, init_zero, 0)

    base = sid * RPT
    for k in range(RPT // CH):
        pltpu.sync_copy(rows0, acc.at[pl.ds(base + k * CH, CH)])
    plsc.subcore_barrier()

    bufs = ((rows0, gsem0, ssem0), (rows1, gsem1, ssem1), (rows2, gsem2, ssem2))

    def scale(ci, rows):
        def group(gi, c2):
            wp = wbuf[ci, pl.ds(gi * 16, 16)]

            @plsc.parallel_loop(0, 16, unroll=4)
            def _edge(i):
                e = gi * 16 + i
                lanes = jnp.broadcast_to(i, (16,)).astype(_i32)
                ws = wp.at[lanes].get(mode="promise_in_bounds")
                for j in range(H // 16):
                    rows[e, pl.ds(j * 16, 16)] = rows[e, pl.ds(j * 16, 16)] * ws
            return c2
        lax.fori_loop(0, CH // 16, group, 0)

    pltpu.async_copy(y.at[sbuf.at[0]], rows0, gsem0)
    pltpu.async_copy(y.at[sbuf.at[1]], rows1, gsem1)

    pltpu.make_async_copy(y.at[sbuf.at[0]], rows0, gsem0).wait()
    scale(0, rows0)
    pltpu.async_copy(rows0, acc.at[dbuf.at[0]], ssem0, add=True)
    pltpu.async_copy(y.at[sbuf.at[2]], rows2, gsem2)

    def chunk_triple(ct, carry):
        for j in range(3):
            c = ct * 3 + j + 1
            rows, gsem, ssem = bufs[(j + 1) % 3]
            prows, pgsem, pssem = bufs[j % 3]
            pltpu.make_async_copy(y.at[sbuf.at[c]], rows, gsem).wait()
            scale(c, rows)
            pltpu.async_copy(rows, acc.at[dbuf.at[c]], ssem, add=True)
            pltpu.make_async_copy(prows, acc.at[dbuf.at[0]], pssem).wait()
            nxt = jnp.where(c + 2 < NCHUNK, c + 2, 0)
            pltpu.async_copy(y.at[sbuf.at[nxt]], prows, pgsem)
        return carry
    lax.fori_loop(0, (NCHUNK - 1) // 3, chunk_triple, 0)

    pltpu.make_async_copy(y.at[sbuf.at[0]], rows1, gsem1).wait()
    pltpu.make_async_copy(y.at[sbuf.at[0]], rows2, gsem2).wait()
    pltpu.make_async_copy(rows0, acc.at[dbuf.at[0]], ssem0).wait()
    plsc.subcore_barrier()

    @pl.when(cid == 0)
    def _():
        pltpu.sync_copy(acc.at[pl.ds(base, RPT)], out0.at[pl.ds(base, RPT)])

    @pl.when(cid == 1)
    def _():
        pltpu.sync_copy(acc.at[pl.ds(base, RPT)], out1.at[pl.ds(base, RPT)])


def _norm_body(d0, d1, d2, d3, o0, o1, o2, o3):
    for dref, oref in ((d0, o0), (d1, o1), (d2, o2), (d3, o3)):
        oref[...] = lax.rsqrt(jnp.maximum(dref[...], 1.0))


def _mm1_body(x_ref, w_ref, o_ref):
    o_ref[...] = jnp.dot(x_ref[...], w_ref[...], preferred_element_type=_f32)


def _mlp_body(a0, a1, wl1a, wl1b, b1, wl2, b2, w2, o_ref):
    h0 = jnp.maximum(a0[...], 0.0)
    h1 = jnp.maximum(a1[...], 0.0)
    z = jnp.dot(h0, wl1a[...], preferred_element_type=_f32)
    z = z + jnp.dot(h1, wl1b[...], preferred_element_type=_f32) + b1[...]
    z = jnp.maximum(z, 0.0)
    z = jnp.dot(z, wl2[...], preferred_element_type=_f32) + b2[...]
    z = jnp.maximum(z, 0.0)
    o_ref[...] = jnp.dot(z, w2[...], preferred_element_type=_f32)


def _out_body(a0, a1, wl3a, wl3b, b3, o_ref):
    h0 = jnp.maximum(a0[...], 0.0)
    h1 = jnp.maximum(a1[...], 0.0)
    z = jnp.dot(h0, wl3a[...], preferred_element_type=_f32)
    o_ref[...] = z + jnp.dot(h1, wl3b[...], preferred_element_type=_f32) + b3[...]


_RB = 1000


def _full(shape):
    return pl.BlockSpec(shape, lambda i: (0, 0))


def _rows(shape):
    return pl.BlockSpec(shape, lambda i: (i, 0))


def _pad_edges(idx_or_w, pad_value):
    flat = jnp.pad(idx_or_w, (0, EPG - E), constant_values=pad_value)
    return flat.reshape(NSUB, NCHUNK, CH)


def kernel(in_feat, edge_index_g0, edge_weight_g0, edge_index_g1, edge_weight_g1,
           W1, W2, Wl1, bl1, Wl2, bl2, Wl3, bl3):
    x = in_feat.astype(_f32)
    s0, d0 = edge_index_g0[0], edge_index_g0[1]
    s1, d1 = edge_index_g1[0], edge_index_g1[1]

    pad_row = N + 8
    sA0 = _pad_edges(s0, pad_row)
    sA1 = _pad_edges(s1, pad_row)
    sB0 = _pad_edges(s0, 0)
    sB1 = _pad_edges(s1, 0)
    dP0 = _pad_edges(d0, pad_row)
    dP1 = _pad_edges(d1, pad_row)
    wP0 = _pad_edges(edge_weight_g0, 0.0)
    wP1 = _pad_edges(edge_weight_g1, 0.0)

    dt0, dt1 = _deg_kernel(sA0, dP0, sA1, dP1)

    degs = [t[:, l].reshape(NP // 128, 128)
            for t in (dt0, dt1) for l in (0, 8)]
    degs = [degs[0], degs[1], degs[2], degs[3]]
    norms = pl.pallas_call(
        _norm_body,
        out_shape=[_sds((NP // 128, 128))] * 4,
    )(*degs)
    ns0, nd0, ns1, nd1 = [t.reshape(NP) for t in norms]

    wf0, wf1 = _fold_kernel(sB0, dP0, wP0, sB1, dP1, wP1, ns0, nd0, ns1, nd1)

    y1 = pl.pallas_call(
        _mm1_body,
        grid=(N // _RB,),
        in_specs=[_rows((_RB, D)), _full((D, H))],
        out_specs=_rows((_RB, H)),
        out_shape=_sds((N, H)),
    )(x, W1)

    agg0, agg1 = _conv_kernel(y1, sB0, dP0, wf0, sB1, dP1, wf1)

    y2 = pl.pallas_call(
        _mlp_body,
        grid=(N // _RB,),
        in_specs=[_rows((_RB, H)), _rows((_RB, H)),
                  _full((H, 2 * H)), _full((H, 2 * H)), _full((1, 2 * H)),
                  _full((2 * H, 2 * H)), _full((1, 2 * H)), _full((2 * H, H))],
        out_specs=_rows((_RB, H)),
        out_shape=_sds((N, H)),
    )(agg0[:N], agg1[:N], Wl1[:H], Wl1[H:], bl1.reshape(1, 2 * H),
      Wl2, bl2.reshape(1, 2 * H), W2)

    agg0b, agg1b = _conv_kernel(y2, sB0, dP0, wf0, sB1, dP1, wf1)

    out = pl.pallas_call(
        _out_body,
        grid=(N // _RB,),
        in_specs=[_rows((_RB, H)), _rows((_RB, H)),
                  _full((H, C)), _full((H, C)), _full((1, C))],
        out_specs=_rows((_RB, C)),
        out_shape=_sds((N, C)),
    )(agg0b[:N], agg1b[:N], Wl3[:H], Wl3[H:], bl3.reshape(1, C))
    return out


# speedup vs baseline: 1.3287x; 1.3287x over previous
import functools

import jax
import jax.numpy as jnp
from jax import lax
from jax.experimental import pallas as pl
from jax.experimental.pallas import tpu as pltpu
from jax.experimental.pallas import tpu_sc as plsc

N = 10000
NP = 10240
E = 320000
D = 128
H = 64
C = 40

NSUB = 16
NCORE = 2
CH = 128
NCHUNK = 158
EPT = NCHUNK * CH
EPG = NSUB * EPT
RPT = NP // NSUB

_f32 = jnp.float32
_i32 = jnp.int32

_MESH = plsc.VectorSubcoreMesh(core_axis_name="c", subcore_axis_name="s",
                               num_cores=NCORE, num_subcores=NSUB)


def _sds(shape, dtype=_f32):
    return jax.ShapeDtypeStruct(shape, dtype)


@functools.partial(
    pl.kernel,
    out_type=[pltpu.HBM((NP, 16), _f32)] * 2,
    mesh=_MESH,
    compiler_params=pltpu.CompilerParams(use_tc_tiling_on_sc=False,
                                         needs_layout_passes=False),
    scratch_types=[
        pltpu.VMEM((NCHUNK, CH), _i32),
        pltpu.VMEM((NCHUNK, CH), _i32),
        pltpu.VMEM((CH, 16), _f32),
        pltpu.VMEM((CH, 16), _f32),
        pltpu.VMEM((RPT, 16), _f32),
        pltpu.VMEM_SHARED((NP, 16), _f32),
    ],
)
def _deg_kernel(src0, dst0, src1, dst1, dtab0, dtab1,
                sbuf, dbuf, ones_s, ones_d, obuf, acc):
    cid = lax.axis_index("c")
    sid = lax.axis_index("s")

    @pl.when(cid == 0)
    def _():
        pltpu.sync_copy(src0.at[sid], sbuf)
        pltpu.sync_copy(dst0.at[sid], dbuf)

    @pl.when(cid == 1)
    def _():
        pltpu.sync_copy(src1.at[sid], sbuf)
        pltpu.sync_copy(dst1.at[sid], dbuf)

    lanes = lax.iota(_i32, 16)
    pat_s = jnp.where(lanes < 8, 1.0, 0.0).astype(_f32)
    pat_d = jnp.where(lanes < 8, 0.0, 1.0).astype(_f32)
    zero = jnp.zeros((16,), _f32)

    def init_ones(i, carry):
        ones_s[i, :] = pat_s
        ones_d[i, :] = pat_d
        return carry
    lax.fori_loop(0, CH, init_ones, 0)

    def init_zero(i, carry):
        obuf[i, :] = zero
        return carry
    lax.fori_loop(0, RPT, init_zero, 0)

    base = sid * RPT
    pltpu.sync_copy(obuf, acc.at[pl.ds(base, RPT)])
    plsc.subcore_barrier()

    def chunk(ci, carry):
        pltpu.sync_copy(ones_s, acc.at[sbuf.at[ci]], add=True)
        pltpu.sync_copy(ones_d, acc.at[dbuf.at[ci]], add=True)
        return carry
    lax.fori_loop(0, NCHUNK, chunk, 0)
    plsc.subcore_barrier()

    pltpu.sync_copy(acc.at[pl.ds(base, RPT)], obuf)

    @pl.when(cid == 0)
    def _():
        pltpu.sync_copy(obuf, dtab0.at[pl.ds(base, RPT)])

    @pl.when(cid == 1)
    def _():
        pltpu.sync_copy(obuf, dtab1.at[pl.ds(base, RPT)])


@functools.partial(
    pl.kernel,
    out_type=[pltpu.HBM((NSUB, NCHUNK, CH), _f32)] * 2,
    mesh=_MESH,
    compiler_params=pltpu.CompilerParams(use_tc_tiling_on_sc=False,
                                         needs_layout_passes=False),
    scratch_types=[
        pltpu.VMEM((NP,), _f32),
        pltpu.VMEM((NP,), _f32),
        pltpu.VMEM((NCHUNK, CH), _i32),
        pltpu.VMEM((NCHUNK, CH), _i32),
        pltpu.VMEM((NCHUNK, CH), _f32),
    ],
)
def _fold_kernel(src0, dst0, w0, src1, dst1, w1, ns0, nd0, ns1, nd1,
                 wp0, wp1, ns_t, nd_t, sbuf, dbuf, wbuf):
    cid = lax.axis_index("c")
    sid = lax.axis_index("s")

    @pl.when(cid == 0)
    def _():
        pltpu.sync_copy(ns0, ns_t)
        pltpu.sync_copy(nd0, nd_t)
        pltpu.sync_copy(src0.at[sid], sbuf)
        pltpu.sync_copy(dst0.at[sid], dbuf)
        pltpu.sync_copy(w0.at[sid], wbuf)

    @pl.when(cid == 1)
    def _():
        pltpu.sync_copy(ns1, ns_t)
        pltpu.sync_copy(nd1, nd_t)
        pltpu.sync_copy(src1.at[sid], sbuf)
        pltpu.sync_copy(dst1.at[sid], dbuf)
        pltpu.sync_copy(w1.at[sid], wbuf)

    def chunk(ci, carry):
        def group(gi, c2):
            sv = sbuf[ci, pl.ds(gi * 16, 16)]
            dv = dbuf[ci, pl.ds(gi * 16, 16)]
            wv = wbuf[ci, pl.ds(gi * 16, 16)]
            nsv = plsc.load_gather(ns_t, [sv])
            ndv = plsc.load_gather(nd_t, [dv])
            wbuf[ci, pl.ds(gi * 16, 16)] = wv * nsv * ndv
            return c2
        lax.fori_loop(0, CH // 16, group, 0)
        return carry
    lax.fori_loop(0, NCHUNK, chunk, 0)

    @pl.when(cid == 0)
    def _():
        pltpu.sync_copy(wbuf, wp0.at[sid])

    @pl.when(cid == 1)
    def _():
        pltpu.sync_copy(wbuf, wp1.at[sid])


@functools.partial(
    pl.kernel,
    out_type=[pltpu.HBM((NP, H), _f32)] * 2,
    mesh=_MESH,
    compiler_params=pltpu.CompilerParams(use_tc_tiling_on_sc=False,
                                         needs_layout_passes=False),
    scratch_types=[
        pltpu.VMEM((NCHUNK, CH), _i32),
        pltpu.VMEM((NCHUNK, CH), _i32),
        pltpu.VMEM((NCHUNK, CH), _f32),
        pltpu.VMEM((CH, H), _f32),
        pltpu.VMEM((CH, H), _f32),
        pltpu.VMEM_SHARED((NP, H), _f32),
        pltpu.SemaphoreType.DMA,
        pltpu.SemaphoreType.DMA,
    ],
)
def _conv_kernel(y, src0, dst0, w0, src1, dst1, w1,
                 out0, out1,
                 sbuf, dbuf, wbuf, rows0, rows1, acc, gsem0, gsem1):
    cid = lax.axis_index("c")
    sid = lax.axis_index("s")

    @pl.when(cid == 0)
    def _():
        pltpu.sync_copy(src0.at[sid], sbuf)
        pltpu.sync_copy(dst0.at[sid], dbuf)
        pltpu.sync_copy(w0.at[sid], wbuf)

    @pl.when(cid == 1)
    def _():
        pltpu.sync_copy(src1.at[sid], sbuf)
        pltpu.sync_copy(dst1.at[sid], dbuf)
        pltpu.sync_copy(w1.at[sid], wbuf)

    zero = jnp.zeros((16,), _f32)

    def init_zero(i, carry):
        for j in range(H // 16):
            rows0[i, pl.ds(j * 16, 16)] = zero
        return carry
    lax.fori_loop(0, CH, init_zero, 0)

    base = sid * RPT
    for k in range(RPT // CH):
        pltpu.sync_copy(rows0, acc.at[pl.ds(base + k * CH, CH)])
    plsc.subcore_barrier()

    pltpu.async_copy(y.at[sbuf.at[0]], rows0, gsem0)
    pltpu.async_copy(y.at[sbuf.at[1]], rows1, gsem1)

    def scale_scatter(ci, rows):
        def group(gi, c2):
            wp = wbuf[ci, pl.ds(gi * 16, 16)]

            @plsc.parallel_loop(0, 16, unroll=8)
            def _edge(i):
                e = gi * 16 + i
                lanes = jnp.broadcast_to(i, (16,)).astype(_i32)
                ws = wp.at[lanes].get(mode="promise_in_bounds")
                for j in range(H // 16):
                    rows[e, pl.ds(j * 16, 16)] = rows[e, pl.ds(j * 16, 16)] * ws
            return c2
        lax.fori_loop(0, CH // 16, group, 0)
        pltpu.sync_copy(rows, acc.at[dbuf.at[ci]], add=True)

    def chunk_pair(cp, carry):
        ci = cp * 2
        for b, (rows, gsem) in enumerate(((rows0, gsem0), (rows1, gsem1))):
            c = ci + b
            pltpu.make_async_copy(y.at[sbuf.at[c]], rows, gsem).wait()
            scale_scatter(c, rows)
            nxt = jnp.where(c + 2 < NCHUNK, c + 2, 0)
            pltpu.async_copy(y.at[sbuf.at[nxt]], rows, gsem)
        return carry
    lax.fori_loop(0, NCHUNK // 2, chunk_pair, 0)
    pltpu.make_async_copy(y.at[sbuf.at[0]], rows0, gsem0).wait()
    pltpu.make_async_copy(y.at[sbuf.at[0]], rows1, gsem1).wait()
    plsc.subcore_barrier()

    @pl.when(cid == 0)
    def _():
        pltpu.sync_copy(acc.at[pl.ds(base, RPT)], out0.at[pl.ds(base, RPT)])

    @pl.when(cid == 1)
    def _():
        pltpu.sync_copy(acc.at[pl.ds(base, RPT)], out1.at[pl.ds(base, RPT)])


def _norm_body(d0, d1, d2, d3, o0, o1, o2, o3):
    for dref, oref in ((d0, o0), (d1, o1), (d2, o2), (d3, o3)):
        oref[...] = lax.rsqrt(jnp.maximum(dref[...], 1.0))


def _mm1_body(x_ref, w_ref, o_ref):
    o_ref[...] = jnp.dot(x_ref[...], w_ref[...], preferred_element_type=_f32)


def _mlp_body(a0, a1, wl1a, wl1b, b1, wl2, b2, w2, o_ref):
    h0 = jnp.maximum(a0[...], 0.0)
    h1 = jnp.maximum(a1[...], 0.0)
    z = jnp.dot(h0, wl1a[...], preferred_element_type=_f32)
    z = z + jnp.dot(h1, wl1b[...], preferred_element_type=_f32) + b1[...]
    z = jnp.maximum(z, 0.0)
    z = jnp.dot(z, wl2[...], preferred_element_type=_f32) + b2[...]
    z = jnp.maximum(z, 0.0)
    o_ref[...] = jnp.dot(z, w2[...], preferred_element_type=_f32)


def _out_body(a0, a1, wl3a, wl3b, b3, o_ref):
    h0 = jnp.maximum(a0[...], 0.0)
    h1 = jnp.maximum(a1[...], 0.0)
    z = jnp.dot(h0, wl3a[...], preferred_element_type=_f32)
    o_ref[...] = z + jnp.dot(h1, wl3b[...], preferred_element_type=_f32) + b3[...]


_RB = 1000


def _full(shape):
    return pl.BlockSpec(shape, lambda i: (0, 0))


def _rows(shape):
    return pl.BlockSpec(shape, lambda i: (i, 0))


def _pad_edges(idx_or_w, pad_value):
    flat = jnp.pad(idx_or_w, (0, EPG - E), constant_values=pad_value)
    return flat.reshape(NSUB, NCHUNK, CH)


def kernel(in_feat, edge_index_g0, edge_weight_g0, edge_index_g1, edge_weight_g1,
           W1, W2, Wl1, bl1, Wl2, bl2, Wl3, bl3):
    x = in_feat.astype(_f32)
    s0, d0 = edge_index_g0[0], edge_index_g0[1]
    s1, d1 = edge_index_g1[0], edge_index_g1[1]

    pad_row = N + 8
    sA0 = _pad_edges(s0, pad_row)
    sA1 = _pad_edges(s1, pad_row)
    sB0 = _pad_edges(s0, 0)
    sB1 = _pad_edges(s1, 0)
    dP0 = _pad_edges(d0, pad_row)
    dP1 = _pad_edges(d1, pad_row)
    wP0 = _pad_edges(edge_weight_g0, 0.0)
    wP1 = _pad_edges(edge_weight_g1, 0.0)

    dt0, dt1 = _deg_kernel(sA0, dP0, sA1, dP1)

    degs = [t[:, l].reshape(NP // 128, 128)
            for t in (dt0, dt1) for l in (0, 8)]
    degs = [degs[0], degs[1], degs[2], degs[3]]
    norms = pl.pallas_call(
        _norm_body,
        out_shape=[_sds((NP // 128, 128))] * 4,
    )(*degs)
    ns0, nd0, ns1, nd1 = [t.reshape(NP) for t in norms]

    wf0, wf1 = _fold_kernel(sB0, dP0, wP0, sB1, dP1, wP1, ns0, nd0, ns1, nd1)

    y1 = pl.pallas_call(
        _mm1_body,
        grid=(N // _RB,),
        in_specs=[_rows((_RB, D)), _full((D, H))],
        out_specs=_rows((_RB, H)),
        out_shape=_sds((N, H)),
    )(x, W1)

    agg0, agg1 = _conv_kernel(y1, sB0, dP0, wf0, sB1, dP1, wf1)

    y2 = pl.pallas_call(
        _mlp_body,
        grid=(N // _RB,),
        in_specs=[_rows((_RB, H)), _rows((_RB, H)),
                  _full((H, 2 * H)), _full((H, 2 * H)), _full((1, 2 * H)),
                  _full((2 * H, 2 * H)), _full((1, 2 * H)), _full((2 * H, H))],
        out_specs=_rows((_RB, H)),
        out_shape=_sds((N, H)),
    )(agg0[:N], agg1[:N], Wl1[:H], Wl1[H:], bl1.reshape(1, 2 * H),
      Wl2, bl2.reshape(1, 2 * H), W2)

    agg0b, agg1b = _conv_kernel(y2, sB0, dP0, wf0, sB1, dP1, wf1)

    out = pl.pallas_call(
        _out_body,
        grid=(N // _RB,),
        in_specs=[_rows((_RB, H)), _rows((_RB, H)),
                  _full((H, C)), _full((H, C)), _full((1, C))],
        out_specs=_rows((_RB, C)),
        out_shape=_sds((N, C)),
    )(agg0b[:N], agg1b[:N], Wl3[:H], Wl3[H:], bl3.reshape(1, C))
    return out


# speedup vs baseline: 1.3487x; 1.0150x over previous
import functools

import jax
import jax.numpy as jnp
from jax import lax
from jax.experimental import pallas as pl
from jax.experimental.pallas import tpu as pltpu
from jax.experimental.pallas import tpu_sc as plsc

N = 10000
NP = 10240
E = 320000
D = 128
H = 64
C = 40

NSUB = 16
NCORE = 2
CH = 128
NCHUNK = 158
EPT = NCHUNK * CH
EPG = NSUB * EPT
RPT = NP // NSUB

_f32 = jnp.float32
_i32 = jnp.int32

_MESH = plsc.VectorSubcoreMesh(core_axis_name="c", subcore_axis_name="s",
                               num_cores=NCORE, num_subcores=NSUB)


def _sds(shape, dtype=_f32):
    return jax.ShapeDtypeStruct(shape, dtype)


@functools.partial(
    pl.kernel,
    out_type=[pltpu.HBM((NP, 16), _f32)] * 2,
    mesh=_MESH,
    compiler_params=pltpu.CompilerParams(use_tc_tiling_on_sc=False,
                                         needs_layout_passes=False),
    scratch_types=[
        pltpu.VMEM((NCHUNK, CH), _i32),
        pltpu.VMEM((NCHUNK, CH), _i32),
        pltpu.VMEM((CH, 16), _f32),
        pltpu.VMEM((CH, 16), _f32),
        pltpu.VMEM((RPT, 16), _f32),
        pltpu.VMEM_SHARED((NP, 16), _f32),
    ],
)
def _deg_kernel(src0, dst0, src1, dst1, dtab0, dtab1,
                sbuf, dbuf, ones_s, ones_d, obuf, acc):
    cid = lax.axis_index("c")
    sid = lax.axis_index("s")

    @pl.when(cid == 0)
    def _():
        pltpu.sync_copy(src0.at[sid], sbuf)
        pltpu.sync_copy(dst0.at[sid], dbuf)

    @pl.when(cid == 1)
    def _():
        pltpu.sync_copy(src1.at[sid], sbuf)
        pltpu.sync_copy(dst1.at[sid], dbuf)

    lanes = lax.iota(_i32, 16)
    pat_s = jnp.where(lanes < 8, 1.0, 0.0).astype(_f32)
    pat_d = jnp.where(lanes < 8, 0.0, 1.0).astype(_f32)
    zero = jnp.zeros((16,), _f32)

    def init_ones(i, carry):
        ones_s[i, :] = pat_s
        ones_d[i, :] = pat_d
        return carry
    lax.fori_loop(0, CH, init_ones, 0)

    def init_zero(i, carry):
        obuf[i, :] = zero
        return carry
    lax.fori_loop(0, RPT, init_zero, 0)

    base = sid * RPT
    pltpu.sync_copy(obuf, acc.at[pl.ds(base, RPT)])
    plsc.subcore_barrier()

    def chunk(ci, carry):
        pltpu.sync_copy(ones_s, acc.at[sbuf.at[ci]], add=True)
        pltpu.sync_copy(ones_d, acc.at[dbuf.at[ci]], add=True)
        return carry
    lax.fori_loop(0, NCHUNK, chunk, 0)
    plsc.subcore_barrier()

    pltpu.sync_copy(acc.at[pl.ds(base, RPT)], obuf)

    @pl.when(cid == 0)
    def _():
        pltpu.sync_copy(obuf, dtab0.at[pl.ds(base, RPT)])

    @pl.when(cid == 1)
    def _():
        pltpu.sync_copy(obuf, dtab1.at[pl.ds(base, RPT)])


@functools.partial(
    pl.kernel,
    out_type=[pltpu.HBM((NSUB, NCHUNK, CH), _f32)] * 2,
    mesh=_MESH,
    compiler_params=pltpu.CompilerParams(use_tc_tiling_on_sc=False,
                                         needs_layout_passes=False),
    scratch_types=[
        pltpu.VMEM((NP,), _f32),
        pltpu.VMEM((NP,), _f32),
        pltpu.VMEM((NCHUNK, CH), _i32),
        pltpu.VMEM((NCHUNK, CH), _i32),
        pltpu.VMEM((NCHUNK, CH), _f32),
    ],
)
def _fold_kernel(src0, dst0, w0, src1, dst1, w1, ns0, nd0, ns1, nd1,
                 wp0, wp1, ns_t, nd_t, sbuf, dbuf, wbuf):
    cid = lax.axis_index("c")
    sid = lax.axis_index("s")

    @pl.when(cid == 0)
    def _():
        pltpu.sync_copy(ns0, ns_t)
        pltpu.sync_copy(nd0, nd_t)
        pltpu.sync_copy(src0.at[sid], sbuf)
        pltpu.sync_copy(dst0.at[sid], dbuf)
        pltpu.sync_copy(w0.at[sid], wbuf)

    @pl.when(cid == 1)
    def _():
        pltpu.sync_copy(ns1, ns_t)
        pltpu.sync_copy(nd1, nd_t)
        pltpu.sync_copy(src1.at[sid], sbuf)
        pltpu.sync_copy(dst1.at[sid], dbuf)
        pltpu.sync_copy(w1.at[sid], wbuf)

    def chunk(ci, carry):
        def group(gi, c2):
            sv = sbuf[ci, pl.ds(gi * 16, 16)]
            dv = dbuf[ci, pl.ds(gi * 16, 16)]
            wv = wbuf[ci, pl.ds(gi * 16, 16)]
            nsv = plsc.load_gather(ns_t, [sv])
            ndv = plsc.load_gather(nd_t, [dv])
            wbuf[ci, pl.ds(gi * 16, 16)] = wv * nsv * ndv
            return c2
        lax.fori_loop(0, CH // 16, group, 0)
        return carry
    lax.fori_loop(0, NCHUNK, chunk, 0)

    @pl.when(cid == 0)
    def _():
        pltpu.sync_copy(wbuf, wp0.at[sid])

    @pl.when(cid == 1)
    def _():
        pltpu.sync_copy(wbuf, wp1.at[sid])


@functools.partial(
    pl.kernel,
    out_type=[pltpu.HBM((NP, H), _f32)] * 2,
    mesh=_MESH,
    compiler_params=pltpu.CompilerParams(use_tc_tiling_on_sc=False,
                                         needs_layout_passes=False),
    scratch_types=[
        pltpu.VMEM((NCHUNK, CH), _i32),
        pltpu.VMEM((NCHUNK, CH), _i32),
        pltpu.VMEM((NCHUNK, CH), _f32),
        pltpu.VMEM((CH, H), _f32),
        pltpu.VMEM((CH, H), _f32),
        pltpu.VMEM_SHARED((NP, H), _f32),
        pltpu.SemaphoreType.DMA,
        pltpu.SemaphoreType.DMA,
    ],
)
def _conv_kernel(y, src0, dst0, w0, src1, dst1, w1,
                 out0, out1,
                 sbuf, dbuf, wbuf, rows0, rows1, acc, gsem0, gsem1):
    cid = lax.axis_index("c")
    sid = lax.axis_index("s")

    @pl.when(cid == 0)
    def _():
        pltpu.sync_copy(src0.at[sid], sbuf)
        pltpu.sync_copy(dst0.at[sid], dbuf)
        pltpu.sync_copy(w0.at[sid], wbuf)

    @pl.when(cid == 1)
    def _():
        pltpu.sync_copy(src1.at[sid], sbuf)
        pltpu.sync_copy(dst1.at[sid], dbuf)
        pltpu.sync_copy(w1.at[sid], wbuf)

    zero = jnp.zeros((16,), _f32)

    def init_zero(i, carry):
        for j in range(H // 16):
            rows0[i, pl.ds(j * 16, 16)] = zero
        return carry
    lax.fori_loop(0, CH, init_zero, 0)

    base = sid * RPT
    for k in range(RPT // CH):
        pltpu.sync_copy(rows0, acc.at[pl.ds(base + k * CH, CH)])
    plsc.subcore_barrier()

    pltpu.async_copy(y.at[sbuf.at[0]], rows0, gsem0)
    pltpu.async_copy(y.at[sbuf.at[1]], rows1, gsem1)

    def scale_scatter(ci, rows):
        @plsc.parallel_loop(0, CH // 16)
        def _group(gi):
            wp = wbuf[ci, pl.ds(gi * 16, 16)]

            @plsc.parallel_loop(0, 16, unroll=4)
            def _edge(i):
                e = gi * 16 + i
                lanes = jnp.broadcast_to(i, (16,)).astype(_i32)
                ws = wp.at[lanes].get(mode="promise_in_bounds")
                for j in range(H // 16):
                    rows[e, pl.ds(j * 16, 16)] = rows[e, pl.ds(j * 16, 16)] * ws
        pltpu.sync_copy(rows, acc.at[dbuf.at[ci]], add=True)

    def chunk_pair(cp, carry):
        ci = cp * 2
        for b, (rows, gsem) in enumerate(((rows0, gsem0), (rows1, gsem1))):
            c = ci + b
            pltpu.make_async_copy(y.at[sbuf.at[c]], rows, gsem).wait()
            scale_scatter(c, rows)
            nxt = jnp.where(c + 2 < NCHUNK, c + 2, 0)
            pltpu.async_copy(y.at[sbuf.at[nxt]], rows, gsem)
        return carry
    lax.fori_loop(0, NCHUNK // 2, chunk_pair, 0)
    pltpu.make_async_copy(y.at[sbuf.at[0]], rows0, gsem0).wait()
    pltpu.make_async_copy(y.at[sbuf.at[0]], rows1, gsem1).wait()
    plsc.subcore_barrier()

    @pl.when(cid == 0)
    def _():
        pltpu.sync_copy(acc.at[pl.ds(base, RPT)], out0.at[pl.ds(base, RPT)])

    @pl.when(cid == 1)
    def _():
        pltpu.sync_copy(acc.at[pl.ds(base, RPT)], out1.at[pl.ds(base, RPT)])


def _norm_body(d0, d1, d2, d3, o0, o1, o2, o3):
    for dref, oref in ((d0, o0), (d1, o1), (d2, o2), (d3, o3)):
        oref[...] = lax.rsqrt(jnp.maximum(dref[...], 1.0))


def _mm1_body(x_ref, w_ref, o_ref):
    o_ref[...] = jnp.dot(x_ref[...], w_ref[...], preferred_element_type=_f32)


def _mlp_body(a0, a1, wl1a, wl1b, b1, wl2, b2, w2, o_ref):
    h0 = jnp.maximum(a0[...], 0.0)
    h1 = jnp.maximum(a1[...], 0.0)
    z = jnp.dot(h0, wl1a[...], preferred_element_type=_f32)
    z = z + jnp.dot(h1, wl1b[...], preferred_element_type=_f32) + b1[...]
    z = jnp.maximum(z, 0.0)
    z = jnp.dot(z, wl2[...], preferred_element_type=_f32) + b2[...]
    z = jnp.maximum(z, 0.0)
    o_ref[...] = jnp.dot(z, w2[...], preferred_element_type=_f32)


def _out_body(a0, a1, wl3a, wl3b, b3, o_ref):
    h0 = jnp.maximum(a0[...], 0.0)
    h1 = jnp.maximum(a1[...], 0.0)
    z = jnp.dot(h0, wl3a[...], preferred_element_type=_f32)
    o_ref[...] = z + jnp.dot(h1, wl3b[...], preferred_element_type=_f32) + b3[...]


_RB = 1000


def _full(shape):
    return pl.BlockSpec(shape, lambda i: (0, 0))


def _rows(shape):
    return pl.BlockSpec(shape, lambda i: (i, 0))


def _pad_edges(idx_or_w, pad_value):
    flat = jnp.pad(idx_or_w, (0, EPG - E), constant_values=pad_value)
    return flat.reshape(NSUB, NCHUNK, CH)


def kernel(in_feat, edge_index_g0, edge_weight_g0, edge_index_g1, edge_weight_g1,
           W1, W2, Wl1, bl1, Wl2, bl2, Wl3, bl3):
    x = in_feat.astype(_f32)
    s0, d0 = edge_index_g0[0], edge_index_g0[1]
    s1, d1 = edge_index_g1[0], edge_index_g1[1]

    pad_row = N + 8
    sA0 = _pad_edges(s0, pad_row)
    sA1 = _pad_edges(s1, pad_row)
    sB0 = _pad_edges(s0, 0)
    sB1 = _pad_edges(s1, 0)
    dP0 = _pad_edges(d0, pad_row)
    dP1 = _pad_edges(d1, pad_row)
    wP0 = _pad_edges(edge_weight_g0, 0.0)
    wP1 = _pad_edges(edge_weight_g1, 0.0)

    dt0, dt1 = _deg_kernel(sA0, dP0, sA1, dP1)

    degs = [t[:, l].reshape(NP // 128, 128)
            for t in (dt0, dt1) for l in (0, 8)]
    degs = [degs[0], degs[1], degs[2], degs[3]]
    norms = pl.pallas_call(
        _norm_body,
        out_shape=[_sds((NP // 128, 128))] * 4,
    )(*degs)
    ns0, nd0, ns1, nd1 = [t.reshape(NP) for t in norms]

    wf0, wf1 = _fold_kernel(sB0, dP0, wP0, sB1, dP1, wP1, ns0, nd0, ns1, nd1)

    y1 = pl.pallas_call(
        _mm1_body,
        grid=(N // _RB,),
        in_specs=[_rows((_RB, D)), _full((D, H))],
        out_specs=_rows((_RB, H)),
        out_shape=_sds((N, H)),
    )(x, W1)

    agg0, agg1 = _conv_kernel(y1, sB0, dP0, wf0, sB1, dP1, wf1)

    y2 = pl.pallas_call(
        _mlp_body,
        grid=(N // _RB,),
        in_specs=[_rows((_RB, H)), _rows((_RB, H)),
                  _full((H, 2 * H)), _full((H, 2 * H)), _full((1, 2 * H)),
                  _full((2 * H, 2 * H)), _full((1, 2 * H)), _full((2 * H, H))],
        out_specs=_rows((_RB, H)),
        out_shape=_sds((N, H)),
    )(agg0[:N], agg1[:N], Wl1[:H], Wl1[H:], bl1.reshape(1, 2 * H),
      Wl2, bl2.reshape(1, 2 * H), W2)

    agg0b, agg1b = _conv_kernel(y2, sB0, dP0, wf0, sB1, dP1, wf1)

    out = pl.pallas_call(
        _out_body,
        grid=(N // _RB,),
        in_specs=[_rows((_RB, H)), _rows((_RB, H)),
                  _full((H, C)), _full((H, C)), _full((1, C))],
        out_specs=_rows((_RB, C)),
        out_shape=_sds((N, C)),
    )(agg0b[:N], agg1b[:N], Wl3[:H], Wl3[H:], bl3.reshape(1, C))
    return out


# speedup vs baseline: 1.3558x; 1.0053x over previous
import functools

import jax
import jax.numpy as jnp
from jax import lax
from jax.experimental import pallas as pl
from jax.experimental.pallas import tpu as pltpu
from jax.experimental.pallas import tpu_sc as plsc

N = 10000
NP = 10240
E = 320000
D = 128
H = 64
C = 40

NSUB = 16
NCORE = 2
CH = 128
NCHUNK = 158
EPT = NCHUNK * CH
EPG = NSUB * EPT
RPT = NP // NSUB

_f32 = jnp.float32
_i32 = jnp.int32

_MESH = plsc.VectorSubcoreMesh(core_axis_name="c", subcore_axis_name="s",
                               num_cores=NCORE, num_subcores=NSUB)


def _sds(shape, dtype=_f32):
    return jax.ShapeDtypeStruct(shape, dtype)


@functools.partial(
    pl.kernel,
    out_type=[pltpu.HBM((NP, 16), _f32)] * 2,
    mesh=_MESH,
    compiler_params=pltpu.CompilerParams(use_tc_tiling_on_sc=False,
                                         needs_layout_passes=False),
    scratch_types=[
        pltpu.VMEM((NCHUNK + 1, CH), _i32),
        pltpu.VMEM((NCHUNK + 1, CH), _i32),
        pltpu.VMEM((CH, 16), _f32),
        pltpu.VMEM((CH, 16), _f32),
        pltpu.VMEM((RPT, 16), _f32),
        pltpu.VMEM_SHARED((NP, 16), _f32),
        pltpu.SemaphoreType.DMA,
        pltpu.SemaphoreType.DMA,
        pltpu.SemaphoreType.DMA,
        pltpu.SemaphoreType.DMA,
    ],
)
def _deg_kernel(src0, dst0, src1, dst1, dtab0, dtab1,
                sbuf, dbuf, ones_s, ones_d, obuf, acc,
                ss0, sd0, ss1, sd1):
    cid = lax.axis_index("c")
    sid = lax.axis_index("s")

    @pl.when(cid == 0)
    def _():
        pltpu.sync_copy(src0.at[sid], sbuf.at[pl.ds(0, NCHUNK)])
        pltpu.sync_copy(dst0.at[sid], dbuf.at[pl.ds(0, NCHUNK)])

    @pl.when(cid == 1)
    def _():
        pltpu.sync_copy(src1.at[sid], sbuf.at[pl.ds(0, NCHUNK)])
        pltpu.sync_copy(dst1.at[sid], dbuf.at[pl.ds(0, NCHUNK)])

    discard = jnp.full((16,), N + 16, _i32)
    for k in range(CH // 16):
        sbuf[NCHUNK, pl.ds(k * 16, 16)] = discard
        dbuf[NCHUNK, pl.ds(k * 16, 16)] = discard

    lanes = lax.iota(_i32, 16)
    pat_s = jnp.where(lanes < 8, 1.0, 0.0).astype(_f32)
    pat_d = jnp.where(lanes < 8, 0.0, 1.0).astype(_f32)
    zero = jnp.zeros((16,), _f32)

    def init_ones(i, carry):
        ones_s[i, :] = pat_s
        ones_d[i, :] = pat_d
        return carry
    lax.fori_loop(0, CH, init_ones, 0)

    def init_zero(i, carry):
        obuf[i, :] = zero
        return carry
    lax.fori_loop(0, RPT, init_zero, 0)

    base = sid * RPT
    pltpu.sync_copy(obuf, acc.at[pl.ds(base, RPT)])
    plsc.subcore_barrier()

    sems = ((ss0, sd0), (ss1, sd1))
    for c0, (ss, sd) in enumerate(sems):
        pltpu.async_copy(ones_s, acc.at[sbuf.at[c0]], ss, add=True)
        pltpu.async_copy(ones_d, acc.at[dbuf.at[c0]], sd, add=True)

    def chunk2(cj, carry):
        for b, (ss, sd) in enumerate(sems):
            ci = cj * 2 + b
            pltpu.make_async_copy(ones_s, acc.at[sbuf.at[ci]], ss).wait()
            pltpu.make_async_copy(ones_d, acc.at[dbuf.at[ci]], sd).wait()
            nxt = jnp.minimum(ci + 2, NCHUNK)
            pltpu.async_copy(ones_s, acc.at[sbuf.at[nxt]], ss, add=True)
            pltpu.async_copy(ones_d, acc.at[dbuf.at[nxt]], sd, add=True)
        return carry
    lax.fori_loop(0, NCHUNK // 2, chunk2, 0)
    for ss, sd in sems:
        pltpu.make_async_copy(ones_s, acc.at[sbuf.at[0]], ss).wait()
        pltpu.make_async_copy(ones_d, acc.at[dbuf.at[0]], sd).wait()
    plsc.subcore_barrier()

    pltpu.sync_copy(acc.at[pl.ds(base, RPT)], obuf)

    @pl.when(cid == 0)
    def _():
        pltpu.sync_copy(obuf, dtab0.at[pl.ds(base, RPT)])

    @pl.when(cid == 1)
    def _():
        pltpu.sync_copy(obuf, dtab1.at[pl.ds(base, RPT)])


@functools.partial(
    pl.kernel,
    out_type=[pltpu.HBM((NSUB, NCHUNK, CH), _f32)] * 2,
    mesh=_MESH,
    compiler_params=pltpu.CompilerParams(use_tc_tiling_on_sc=False,
                                         needs_layout_passes=False),
    scratch_types=[
        pltpu.VMEM((NP,), _f32),
        pltpu.VMEM((NP,), _f32),
        pltpu.VMEM((NCHUNK, CH), _i32),
        pltpu.VMEM((NCHUNK, CH), _i32),
        pltpu.VMEM((NCHUNK, CH), _f32),
    ],
)
def _fold_kernel(src0, dst0, w0, src1, dst1, w1, ns0, nd0, ns1, nd1,
                 wp0, wp1, ns_t, nd_t, sbuf, dbuf, wbuf):
    cid = lax.axis_index("c")
    sid = lax.axis_index("s")

    @pl.when(cid == 0)
    def _():
        pltpu.sync_copy(ns0, ns_t)
        pltpu.sync_copy(nd0, nd_t)
        pltpu.sync_copy(src0.at[sid], sbuf)
        pltpu.sync_copy(dst0.at[sid], dbuf)
        pltpu.sync_copy(w0.at[sid], wbuf)

    @pl.when(cid == 1)
    def _():
        pltpu.sync_copy(ns1, ns_t)
        pltpu.sync_copy(nd1, nd_t)
        pltpu.sync_copy(src1.at[sid], sbuf)
        pltpu.sync_copy(dst1.at[sid], dbuf)
        pltpu.sync_copy(w1.at[sid], wbuf)

    def chunk(ci, carry):
        def group(gi, c2):
            sv = sbuf[ci, pl.ds(gi * 16, 16)]
            dv = dbuf[ci, pl.ds(gi * 16, 16)]
            wv = wbuf[ci, pl.ds(gi * 16, 16)]
            nsv = plsc.load_gather(ns_t, [sv])
            ndv = plsc.load_gather(nd_t, [dv])
            wbuf[ci, pl.ds(gi * 16, 16)] = wv * nsv * ndv
            return c2
        lax.fori_loop(0, CH // 16, group, 0)
        return carry
    lax.fori_loop(0, NCHUNK, chunk, 0)

    @pl.when(cid == 0)
    def _():
        pltpu.sync_copy(wbuf, wp0.at[sid])

    @pl.when(cid == 1)
    def _():
        pltpu.sync_copy(wbuf, wp1.at[sid])


@functools.partial(
    pl.kernel,
    out_type=[pltpu.HBM((NP, H), _f32)] * 2,
    mesh=_MESH,
    compiler_params=pltpu.CompilerParams(use_tc_tiling_on_sc=False,
                                         needs_layout_passes=False),
    scratch_types=[
        pltpu.VMEM((NCHUNK, CH), _i32),
        pltpu.VMEM((NCHUNK, CH), _i32),
        pltpu.VMEM((NCHUNK, CH), _f32),
        pltpu.VMEM((CH, H), _f32),
        pltpu.VMEM((CH, H), _f32),
        pltpu.VMEM_SHARED((NP, H), _f32),
        pltpu.SemaphoreType.DMA,
        pltpu.SemaphoreType.DMA,
    ],
)
def _conv_kernel(y, src0, dst0, w0, src1, dst1, w1,
                 out0, out1,
                 sbuf, dbuf, wbuf, rows0, rows1, acc, gsem0, gsem1):
    cid = lax.axis_index("c")
    sid = lax.axis_index("s")

    @pl.when(cid == 0)
    def _():
        pltpu.sync_copy(src0.at[sid], sbuf)
        pltpu.sync_copy(dst0.at[sid], dbuf)
        pltpu.sync_copy(w0.at[sid], wbuf)

    @pl.when(cid == 1)
    def _():
        pltpu.sync_copy(src1.at[sid], sbuf)
        pltpu.sync_copy(dst1.at[sid], dbuf)
        pltpu.sync_copy(w1.at[sid], wbuf)

    zero = jnp.zeros((16,), _f32)

    def init_zero(i, carry):
        for j in range(H // 16):
            rows0[i, pl.ds(j * 16, 16)] = zero
        return carry
    lax.fori_loop(0, CH, init_zero, 0)

    base = sid * RPT
    for k in range(RPT // CH):
        pltpu.sync_copy(rows0, acc.at[pl.ds(base + k * CH, CH)])
    plsc.subcore_barrier()

    pltpu.async_copy(y.at[sbuf.at[0]], rows0, gsem0)
    pltpu.async_copy(y.at[sbuf.at[1]], rows1, gsem1)

    def scale_scatter(ci, rows):
        def group(gi, c2):
            wp = wbuf[ci, pl.ds(gi * 16, 16)]

            @plsc.parallel_loop(0, 16, unroll=4)
            def _edge(i):
                e = gi * 16 + i
                lanes = jnp.broadcast_to(i, (16,)).astype(_i32)
                ws = wp.at[lanes].get(mode="promise_in_bounds")
                for j in range(H // 16):
                    rows[e, pl.ds(j * 16, 16)] = rows[e, pl.ds(j * 16, 16)] * ws
            return c2
        lax.fori_loop(0, CH // 16, group, 0)
        pltpu.sync_copy(rows, acc.at[dbuf.at[ci]], add=True)

    def chunk_pair(cp, carry):
        ci = cp * 2
        for b, (rows, gsem) in enumerate(((rows0, gsem0), (rows1, gsem1))):
            c = ci + b
            pltpu.make_async_copy(y.at[sbuf.at[c]], rows, gsem).wait()
            scale_scatter(c, rows)
            nxt = jnp.where(c + 2 < NCHUNK, c + 2, 0)
            pltpu.async_copy(y.at[sbuf.at[nxt]], rows, gsem)
        return carry
    lax.fori_loop(0, NCHUNK // 2, chunk_pair, 0)
    pltpu.make_async_copy(y.at[sbuf.at[0]], rows0, gsem0).wait()
    pltpu.make_async_copy(y.at[sbuf.at[0]], rows1, gsem1).wait()
    plsc.subcore_barrier()

    @pl.when(cid == 0)
    def _():
        pltpu.sync_copy(acc.at[pl.ds(base, RPT)], out0.at[pl.ds(base, RPT)])

    @pl.when(cid == 1)
    def _():
        pltpu.sync_copy(acc.at[pl.ds(base, RPT)], out1.at[pl.ds(base, RPT)])


def _norm_body(d0, d1, d2, d3, o0, o1, o2, o3):
    for dref, oref in ((d0, o0), (d1, o1), (d2, o2), (d3, o3)):
        oref[...] = lax.rsqrt(jnp.maximum(dref[...], 1.0))


def _mm1_body(x_ref, w_ref, o_ref):
    o_ref[...] = jnp.dot(x_ref[...], w_ref[...], preferred_element_type=_f32)


def _mlp_body(a0, a1, wl1a, wl1b, b1, wl2, b2, w2, o_ref):
    h0 = jnp.maximum(a0[...], 0.0)
    h1 = jnp.maximum(a1[...], 0.0)
    z = jnp.dot(h0, wl1a[...], preferred_element_type=_f32)
    z = z + jnp.dot(h1, wl1b[...], preferred_element_type=_f32) + b1[...]
    z = jnp.maximum(z, 0.0)
    z = jnp.dot(z, wl2[...], preferred_element_type=_f32) + b2[...]
    z = jnp.maximum(z, 0.0)
    o_ref[...] = jnp.dot(z, w2[...], preferred_element_type=_f32)


def _out_body(a0, a1, wl3a, wl3b, b3, o_ref):
    h0 = jnp.maximum(a0[...], 0.0)
    h1 = jnp.maximum(a1[...], 0.0)
    z = jnp.dot(h0, wl3a[...], preferred_element_type=_f32)
    o_ref[...] = z + jnp.dot(h1, wl3b[...], preferred_element_type=_f32) + b3[...]


_RB = 1000


def _full(shape):
    return pl.BlockSpec(shape, lambda i: (0, 0))


def _rows(shape):
    return pl.BlockSpec(shape, lambda i: (i, 0))


def _pad_edges(idx_or_w, pad_value):
    flat = jnp.pad(idx_or_w, (0, EPG - E), constant_values=pad_value)
    return flat.reshape(NSUB, NCHUNK, CH)


def kernel(in_feat, edge_index_g0, edge_weight_g0, edge_index_g1, edge_weight_g1,
           W1, W2, Wl1, bl1, Wl2, bl2, Wl3, bl3):
    x = in_feat.astype(_f32)
    s0, d0 = edge_index_g0[0], edge_index_g0[1]
    s1, d1 = edge_index_g1[0], edge_index_g1[1]

    pad_row = N + 8
    sA0 = _pad_edges(s0, pad_row)
    sA1 = _pad_edges(s1, pad_row)
    sB0 = _pad_edges(s0, 0)
    sB1 = _pad_edges(s1, 0)
    dP0 = _pad_edges(d0, pad_row)
    dP1 = _pad_edges(d1, pad_row)
    wP0 = _pad_edges(edge_weight_g0, 0.0)
    wP1 = _pad_edges(edge_weight_g1, 0.0)

    dt0, dt1 = _deg_kernel(sA0, dP0, sA1, dP1)

    degs = [t[:, l].reshape(NP // 128, 128)
            for t in (dt0, dt1) for l in (0, 8)]
    degs = [degs[0], degs[1], degs[2], degs[3]]
    norms = pl.pallas_call(
        _norm_body,
        out_shape=[_sds((NP // 128, 128))] * 4,
    )(*degs)
    ns0, nd0, ns1, nd1 = [t.reshape(NP) for t in norms]

    wf0, wf1 = _fold_kernel(sB0, dP0, wP0, sB1, dP1, wP1, ns0, nd0, ns1, nd1)

    y1 = pl.pallas_call(
        _mm1_body,
        grid=(N // _RB,),
        in_specs=[_rows((_RB, D)), _full((D, H))],
        out_specs=_rows((_RB, H)),
        out_shape=_sds((N, H)),
    )(x, W1)

    agg0, agg1 = _conv_kernel(y1, sB0, dP0, wf0, sB1, dP1, wf1)

    y2 = pl.pallas_call(
        _mlp_body,
        grid=(N // _RB,),
        in_specs=[_rows((_RB, H)), _rows((_RB, H)),
                  _full((H, 2 * H)), _full((H, 2 * H)), _full((1, 2 * H)),
                  _full((2 * H, 2 * H)), _full((1, 2 * H)), _full((2 * H, H))],
        out_specs=_rows((_RB, H)),
        out_shape=_sds((N, H)),
    )(agg0[:N], agg1[:N], Wl1[:H], Wl1[H:], bl1.reshape(1, 2 * H),
      Wl2, bl2.reshape(1, 2 * H), W2)

    agg0b, agg1b = _conv_kernel(y2, sB0, dP0, wf0, sB1, dP1, wf1)

    out = pl.pallas_call(
        _out_body,
        grid=(N // _RB,),
        in_specs=[_rows((_RB, H)), _rows((_RB, H)),
                  _full((H, C)), _full((H, C)), _full((1, C))],
        out_specs=_rows((_RB, C)),
        out_shape=_sds((N, C)),
    )(agg0b[:N], agg1b[:N], Wl3[:H], Wl3[H:], bl3.reshape(1, C))
    return out


# speedup vs baseline: 1.3755x; 1.0145x over previous
import functools

import jax
import jax.numpy as jnp
from jax import lax
from jax.experimental import pallas as pl
from jax.experimental.pallas import tpu as pltpu
from jax.experimental.pallas import tpu_sc as plsc

N = 10000
NP = 10240
E = 320000
D = 128
H = 64
C = 40

NSUB = 16
NCORE = 2
CH = 128
NCHUNK = 158
EPT = NCHUNK * CH
EPG = NSUB * EPT
RPT = NP // NSUB

_f32 = jnp.float32
_i32 = jnp.int32

_MESH = plsc.VectorSubcoreMesh(core_axis_name="c", subcore_axis_name="s",
                               num_cores=NCORE, num_subcores=NSUB)


def _sds(shape, dtype=_f32):
    return jax.ShapeDtypeStruct(shape, dtype)


@functools.partial(
    pl.kernel,
    out_type=[pltpu.HBM((NSUB, NCHUNK, CH), _f32)] * 2,
    mesh=_MESH,
    compiler_params=pltpu.CompilerParams(use_tc_tiling_on_sc=False,
                                         needs_layout_passes=False),
    scratch_types=[
        pltpu.VMEM((NCHUNK + 1, CH), _i32),
        pltpu.VMEM((NCHUNK + 1, CH), _i32),
        pltpu.VMEM((NCHUNK, CH), _f32),
        pltpu.VMEM((CH, 16), _f32),
        pltpu.VMEM((CH, 16), _f32),
        pltpu.VMEM((RPT, 16), _f32),
        pltpu.VMEM((RPT,), _f32),
        pltpu.VMEM((RPT,), _f32),
        pltpu.VMEM((NP,), _f32),
        pltpu.VMEM((NP,), _f32),
        pltpu.VMEM_SHARED((NP, 16), _f32),
        pltpu.VMEM_SHARED((NP,), _f32),
        pltpu.VMEM_SHARED((NP,), _f32),
        pltpu.SemaphoreType.DMA,
        pltpu.SemaphoreType.DMA,
        pltpu.SemaphoreType.DMA,
        pltpu.SemaphoreType.DMA,
    ],
)
def _degfold_kernel(src0, dst0, w0, src1, dst1, w1, wp0, wp1,
                    sbuf, dbuf, wbuf, ones_s, ones_d, obuf, ns_c, nd_c,
                    ns_t, nd_t, acc, ns_sh, nd_sh, ss0, sd0, ss1, sd1):
    cid = lax.axis_index("c")
    sid = lax.axis_index("s")

    @pl.when(cid == 0)
    def _():
        pltpu.sync_copy(src0.at[sid], sbuf.at[pl.ds(0, NCHUNK)])
        pltpu.sync_copy(dst0.at[sid], dbuf.at[pl.ds(0, NCHUNK)])
        pltpu.sync_copy(w0.at[sid], wbuf)

    @pl.when(cid == 1)
    def _():
        pltpu.sync_copy(src1.at[sid], sbuf.at[pl.ds(0, NCHUNK)])
        pltpu.sync_copy(dst1.at[sid], dbuf.at[pl.ds(0, NCHUNK)])
        pltpu.sync_copy(w1.at[sid], wbuf)

    discard = jnp.full((16,), N + 16, _i32)
    for k in range(CH // 16):
        sbuf[NCHUNK, pl.ds(k * 16, 16)] = discard
        dbuf[NCHUNK, pl.ds(k * 16, 16)] = discard

    lanes = lax.iota(_i32, 16)
    pat_s = jnp.where(lanes < 8, 1.0, 0.0).astype(_f32)
    pat_d = jnp.where(lanes < 8, 0.0, 1.0).astype(_f32)
    zero = jnp.zeros((16,), _f32)

    def init_ones(i, carry):
        ones_s[i, :] = pat_s
        ones_d[i, :] = pat_d
        return carry
    lax.fori_loop(0, CH, init_ones, 0)

    def init_zero(i, carry):
        obuf[i, :] = zero
        return carry
    lax.fori_loop(0, RPT, init_zero, 0)

    base = sid * RPT
    pltpu.sync_copy(obuf, acc.at[pl.ds(base, RPT)])
    plsc.subcore_barrier()

    sems = ((ss0, sd0), (ss1, sd1))
    for c0, (ss, sd) in enumerate(sems):
        pltpu.async_copy(ones_s, acc.at[sbuf.at[c0]], ss, add=True)
        pltpu.async_copy(ones_d, acc.at[dbuf.at[c0]], sd, add=True)

    def chunk2(cj, carry):
        for b, (ss, sd) in enumerate(sems):
            ci = cj * 2 + b
            pltpu.make_async_copy(ones_s, acc.at[sbuf.at[ci]], ss).wait()
            pltpu.make_async_copy(ones_d, acc.at[dbuf.at[ci]], sd).wait()
            nxt = jnp.minimum(ci + 2, NCHUNK)
            pltpu.async_copy(ones_s, acc.at[sbuf.at[nxt]], ss, add=True)
            pltpu.async_copy(ones_d, acc.at[dbuf.at[nxt]], sd, add=True)
        return carry
    lax.fori_loop(0, NCHUNK // 2, chunk2, 0)
    for ss, sd in sems:
        pltpu.make_async_copy(ones_s, acc.at[sbuf.at[0]], ss).wait()
        pltpu.make_async_copy(ones_d, acc.at[dbuf.at[0]], sd).wait()
    plsc.subcore_barrier()

    pltpu.sync_copy(acc.at[pl.ds(base, RPT)], obuf)
    magic = jnp.full((16,), 0x5F3759DF, _i32)
    half, w15 = jnp.full((16,), 0.5, _f32), jnp.full((16,), 1.5, _f32)

    def rsqrt_row(r, carry):
        x = jnp.maximum(obuf[r, :], 1.0)
        y = plsc.bitcast(magic - (plsc.bitcast(x, _i32) >> 1), _f32)
        for _ in range(3):
            y = y * (w15 - half * x * y * y)
        obuf[r, :] = y
        return carry
    lax.fori_loop(0, RPT, rsqrt_row, 0)

    zl = jnp.zeros((16,), _i32)
    el = jnp.full((16,), 8, _i32)

    def compact(t, carry):
        rows = lax.iota(_i32, 16) + t * 16
        ns_c[pl.ds(t * 16, 16)] = plsc.load_gather(obuf, [rows, zl])
        nd_c[pl.ds(t * 16, 16)] = plsc.load_gather(obuf, [rows, el])
        return carry
    lax.fori_loop(0, RPT // 16, compact, 0)
    pltpu.sync_copy(ns_c, ns_sh.at[pl.ds(base, RPT)])
    pltpu.sync_copy(nd_c, nd_sh.at[pl.ds(base, RPT)])
    plsc.subcore_barrier()
    pltpu.sync_copy(ns_sh, ns_t)
    pltpu.sync_copy(nd_sh, nd_t)

    def chunk(ci, carry):
        def group(gi, c2):
            sv = sbuf[ci, pl.ds(gi * 16, 16)]
            dv = dbuf[ci, pl.ds(gi * 16, 16)]
            wv = wbuf[ci, pl.ds(gi * 16, 16)]
            nsv = plsc.load_gather(ns_t, [sv])
            ndv = plsc.load_gather(nd_t, [dv])
            wbuf[ci, pl.ds(gi * 16, 16)] = wv * nsv * ndv
            return c2
        lax.fori_loop(0, CH // 16, group, 0)
        return carry
    lax.fori_loop(0, NCHUNK, chunk, 0)

    @pl.when(cid == 0)
    def _():
        pltpu.sync_copy(wbuf, wp0.at[sid])

    @pl.when(cid == 1)
    def _():
        pltpu.sync_copy(wbuf, wp1.at[sid])


@functools.partial(
    pl.kernel,
    out_type=[pltpu.HBM((NP, H), _f32)] * 2,
    mesh=_MESH,
    compiler_params=pltpu.CompilerParams(use_tc_tiling_on_sc=False,
                                         needs_layout_passes=False),
    scratch_types=[
        pltpu.VMEM((NCHUNK, CH), _i32),
        pltpu.VMEM((NCHUNK, CH), _i32),
        pltpu.VMEM((NCHUNK, CH), _f32),
        pltpu.VMEM((CH, H), _f32),
        pltpu.VMEM((CH, H), _f32),
        pltpu.VMEM_SHARED((NP, H), _f32),
        pltpu.SemaphoreType.DMA,
        pltpu.SemaphoreType.DMA,
    ],
)
def _conv_kernel(y, src0, dst0, w0, src1, dst1, w1,
                 out0, out1,
                 sbuf, dbuf, wbuf, rows0, rows1, acc, gsem0, gsem1):
    cid = lax.axis_index("c")
    sid = lax.axis_index("s")

    @pl.when(cid == 0)
    def _():
        pltpu.sync_copy(src0.at[sid], sbuf)
        pltpu.sync_copy(dst0.at[sid], dbuf)
        pltpu.sync_copy(w0.at[sid], wbuf)

    @pl.when(cid == 1)
    def _():
        pltpu.sync_copy(src1.at[sid], sbuf)
        pltpu.sync_copy(dst1.at[sid], dbuf)
        pltpu.sync_copy(w1.at[sid], wbuf)

    zero = jnp.zeros((16,), _f32)

    def init_zero(i, carry):
        for j in range(H // 16):
            rows0[i, pl.ds(j * 16, 16)] = zero
        return carry
    lax.fori_loop(0, CH, init_zero, 0)

    base = sid * RPT
    for k in range(RPT // CH):
        pltpu.sync_copy(rows0, acc.at[pl.ds(base + k * CH, CH)])
    plsc.subcore_barrier()

    pltpu.async_copy(y.at[sbuf.at[0]], rows0, gsem0)
    pltpu.async_copy(y.at[sbuf.at[1]], rows1, gsem1)

    def scale_scatter(ci, rows):
        def group(gi, c2):
            wp = wbuf[ci, pl.ds(gi * 16, 16)]

            @plsc.parallel_loop(0, 16, unroll=4)
            def _edge(i):
                e = gi * 16 + i
                lanes = jnp.broadcast_to(i, (16,)).astype(_i32)
                ws = wp.at[lanes].get(mode="promise_in_bounds")
                for j in range(H // 16):
                    rows[e, pl.ds(j * 16, 16)] = rows[e, pl.ds(j * 16, 16)] * ws
            return c2
        lax.fori_loop(0, CH // 16, group, 0)
        pltpu.sync_copy(rows, acc.at[dbuf.at[ci]], add=True)

    def chunk_pair(cp, carry):
        ci = cp * 2
        for b, (rows, gsem) in enumerate(((rows0, gsem0), (rows1, gsem1))):
            c = ci + b
            pltpu.make_async_copy(y.at[sbuf.at[c]], rows, gsem).wait()
            scale_scatter(c, rows)
            nxt = jnp.where(c + 2 < NCHUNK, c + 2, 0)
            pltpu.async_copy(y.at[sbuf.at[nxt]], rows, gsem)
        return carry
    lax.fori_loop(0, NCHUNK // 2, chunk_pair, 0)
    pltpu.make_async_copy(y.at[sbuf.at[0]], rows0, gsem0).wait()
    pltpu.make_async_copy(y.at[sbuf.at[0]], rows1, gsem1).wait()
    plsc.subcore_barrier()

    @pl.when(cid == 0)
    def _():
        pltpu.sync_copy(acc.at[pl.ds(base, RPT)], out0.at[pl.ds(base, RPT)])

    @pl.when(cid == 1)
    def _():
        pltpu.sync_copy(acc.at[pl.ds(base, RPT)], out1.at[pl.ds(base, RPT)])


def _mm1_body(x_ref, w_ref, o_ref):
    o_ref[...] = jnp.dot(x_ref[...], w_ref[...], preferred_element_type=_f32)


def _mlp_body(a0, a1, wl1a, wl1b, b1, wl2, b2, w2, o_ref):
    h0 = jnp.maximum(a0[...], 0.0)
    h1 = jnp.maximum(a1[...], 0.0)
    z = jnp.dot(h0, wl1a[...], preferred_element_type=_f32)
    z = z + jnp.dot(h1, wl1b[...], preferred_element_type=_f32) + b1[...]
    z = jnp.maximum(z, 0.0)
    z = jnp.dot(z, wl2[...], preferred_element_type=_f32) + b2[...]
    z = jnp.maximum(z, 0.0)
    o_ref[...] = jnp.dot(z, w2[...], preferred_element_type=_f32)


def _out_body(a0, a1, wl3a, wl3b, b3, o_ref):
    h0 = jnp.maximum(a0[...], 0.0)
    h1 = jnp.maximum(a1[...], 0.0)
    z = jnp.dot(h0, wl3a[...], preferred_element_type=_f32)
    o_ref[...] = z + jnp.dot(h1, wl3b[...], preferred_element_type=_f32) + b3[...]


_RB = 1000


def _full(shape):
    return pl.BlockSpec(shape, lambda i: (0, 0))


def _rows(shape):
    return pl.BlockSpec(shape, lambda i: (i, 0))


def _pad_edges(idx_or_w, pad_value):
    flat = jnp.pad(idx_or_w, (0, EPG - E), constant_values=pad_value)
    return flat.reshape(NSUB, NCHUNK, CH)


def kernel(in_feat, edge_index_g0, edge_weight_g0, edge_index_g1, edge_weight_g1,
           W1, W2, Wl1, bl1, Wl2, bl2, Wl3, bl3):
    x = in_feat.astype(_f32)
    s0, d0 = edge_index_g0[0], edge_index_g0[1]
    s1, d1 = edge_index_g1[0], edge_index_g1[1]

    pad_row = N + 8
    sA0 = _pad_edges(s0, pad_row)
    sA1 = _pad_edges(s1, pad_row)
    sB0 = _pad_edges(s0, 0)
    sB1 = _pad_edges(s1, 0)
    dP0 = _pad_edges(d0, pad_row)
    dP1 = _pad_edges(d1, pad_row)
    wP0 = _pad_edges(edge_weight_g0, 0.0)
    wP1 = _pad_edges(edge_weight_g1, 0.0)

    wf0, wf1 = _degfold_kernel(sA0, dP0, wP0, sA1, dP1, wP1)

    y1 = pl.pallas_call(
        _mm1_body,
        grid=(N // _RB,),
        in_specs=[_rows((_RB, D)), _full((D, H))],
        out_specs=_rows((_RB, H)),
        out_shape=_sds((N, H)),
    )(x, W1)

    agg0, agg1 = _conv_kernel(y1, sB0, dP0, wf0, sB1, dP1, wf1)

    y2 = pl.pallas_call(
        _mlp_body,
        grid=(N // _RB,),
        in_specs=[_rows((_RB, H)), _rows((_RB, H)),
                  _full((H, 2 * H)), _full((H, 2 * H)), _full((1, 2 * H)),
                  _full((2 * H, 2 * H)), _full((1, 2 * H)), _full((2 * H, H))],
        out_specs=_rows((_RB, H)),
        out_shape=_sds((N, H)),
    )(agg0[:N], agg1[:N], Wl1[:H], Wl1[H:], bl1.reshape(1, 2 * H),
      Wl2, bl2.reshape(1, 2 * H), W2)

    agg0b, agg1b = _conv_kernel(y2, sB0, dP0, wf0, sB1, dP1, wf1)

    out = pl.pallas_call(
        _out_body,
        grid=(N // _RB,),
        in_specs=[_rows((_RB, H)), _rows((_RB, H)),
                  _full((H, C)), _full((H, C)), _full((1, C))],
        out_specs=_rows((_RB, C)),
        out_shape=_sds((N, C)),
    )(agg0b[:N], agg1b[:N], Wl3[:H], Wl3[H:], bl3.reshape(1, C))
    return out
